# Initial kernel scaffold; baseline (speedup 1.0000x reference)
import jax
import jax.numpy as jnp
from jax.experimental import pallas as pl


def kernel(x, grid, W_mat, b):
    raise NotImplementedError("write your pallas kernel here")


# speedup vs baseline: 29.6903x; 29.6903x over previous
import functools

import jax
import jax.numpy as jnp
from jax import lax
from jax.experimental import pallas as pl
from jax.experimental.pallas import tpu as pltpu
from jax.experimental.pallas import tpu_sc as plsc

_H = 256
_WG = 192
_F = 64
_HW = _H * _WG
_N = 1048576
_NC = 2
_NS = 16
_NW = _NC * _NS
_PPW = _N // _NW
_CHUNK = 2048
_NCHUNK = _PPW // _CHUNK
_VPC = _CHUNK // 16


def _proj_body(g_ref, w_ref, b_ref, p_ref):
    p_ref[...] = (
        jnp.dot(g_ref[...], w_ref[...], preferred_element_type=jnp.float32)
        + b_ref[...]
    )


def _project(grid2d, w_pad, b_pad):
    return pl.pallas_call(
        _proj_body,
        out_shape=jax.ShapeDtypeStruct((_HW, 8), jnp.float32),
    )(grid2d, w_pad, b_pad)


def _interp_body(x_hbm, pa_hbm, pb_hbm, out_hbm, pa_v, pb_v, xs_v, os_v):
    wid = lax.axis_index("s") * _NC + lax.axis_index("c")
    pltpu.sync_copy(pa_hbm, pa_v)
    pltpu.sync_copy(pb_hbm, pb_v)
    base = wid * _PPW
    lane = lax.iota(jnp.int32, 16)
    hmask = jnp.int32(-65536)

    def vec_body(v, _):
        ix = (v * 16 + lane) * 2
        xv = plsc.load_gather(xs_v, [ix])
        yv = plsc.load_gather(xs_v, [ix + 1])
        gx = xv * jnp.float32(_H - 1)
        gy = yv * jnp.float32(_WG - 1)
        x0 = jnp.minimum(gx.astype(jnp.int32), _H - 2)
        y0 = jnp.minimum(gy.astype(jnp.int32), _WG - 2)
        wx = gx - x0.astype(jnp.float32)
        wy = gy - y0.astype(jnp.float32)
        i00 = x0 * _WG + y0
        i01 = i00 + 1
        i10 = i00 + _WG
        i11 = i00 + _WG + 1
        a00 = plsc.load_gather(pa_v, [i00])
        a01 = plsc.load_gather(pa_v, [i01])
        a10 = plsc.load_gather(pa_v, [i10])
        a11 = plsc.load_gather(pa_v, [i11])
        b00 = plsc.load_gather(pb_v, [i00])
        b01 = plsc.load_gather(pb_v, [i01])
        b10 = plsc.load_gather(pb_v, [i10])
        b11 = plsc.load_gather(pb_v, [i11])
        u = 1.0 - wx
        t = 1.0 - wy
        w00 = u * t
        w10 = wx * t
        w01 = u * wy
        w11 = wx * wy
        o0 = (w00 * plsc.bitcast(a00 << 16, jnp.float32)
              + w10 * plsc.bitcast(a10 << 16, jnp.float32)
              + w01 * plsc.bitcast(a01 << 16, jnp.float32)
              + w11 * plsc.bitcast(a11 << 16, jnp.float32))
        o1 = (w00 * plsc.bitcast(a00 & hmask, jnp.float32)
              + w10 * plsc.bitcast(a10 & hmask, jnp.float32)
              + w01 * plsc.bitcast(a01 & hmask, jnp.float32)
              + w11 * plsc.bitcast(a11 & hmask, jnp.float32))
        o2 = w00 * b00 + w10 * b10 + w01 * b01 + w11 * b11
        si = (v * 16 + lane) * 3
        plsc.store_scatter(os_v, [si], o0)
        plsc.store_scatter(os_v, [si + 1], o1)
        plsc.store_scatter(os_v, [si + 2], o2)
        return 0

    def chunk_body(j, _):
        cbase = base + j * _CHUNK
        pltpu.sync_copy(x_hbm.at[pl.ds(cbase * 2, _CHUNK * 2)], xs_v)
        lax.fori_loop(0, _VPC, vec_body, 0)
        pltpu.sync_copy(os_v, out_hbm.at[pl.ds(cbase * 3, _CHUNK * 3)])
        return 0

    lax.fori_loop(0, _NCHUNK, chunk_body, 0)


_interp = functools.partial(
    pl.kernel,
    out_type=jax.ShapeDtypeStruct((_N * 3,), jnp.float32),
    mesh=plsc.VectorSubcoreMesh(
        core_axis_name="c", subcore_axis_name="s",
        num_cores=_NC, num_subcores=_NS,
    ),
    scratch_types=[
        pltpu.VMEM((_HW,), jnp.int32),
        pltpu.VMEM((_HW,), jnp.float32),
        pltpu.VMEM((_CHUNK * 2,), jnp.float32),
        pltpu.VMEM((_CHUNK * 3,), jnp.float32),
    ],
    compiler_params=pltpu.CompilerParams(needs_layout_passes=False),
)(_interp_body)


def kernel(x, grid, W_mat, b):
    grid2d = grid.reshape(_HW, _F)
    w_pad = jnp.zeros((_F, 8), jnp.float32).at[:, :3].set(W_mat)
    b_pad = jnp.zeros((1, 8), jnp.float32).at[0, :3].set(b)
    p = _project(grid2d, w_pad, b_pad)
    u0 = lax.bitcast_convert_type(
        p[:, 0].astype(jnp.bfloat16), jnp.uint16).astype(jnp.uint32)
    u1 = lax.bitcast_convert_type(
        p[:, 1].astype(jnp.bfloat16), jnp.uint16).astype(jnp.uint32)
    plane_a = lax.bitcast_convert_type((u1 << 16) | u0, jnp.int32)
    out_flat = _interp(x.reshape(_N * 2), plane_a, p[:, 2])
    return (out_flat.reshape(_N, 3), x)


# speedup vs baseline: 30.2634x; 1.0193x over previous
import functools

import jax
import jax.numpy as jnp
from jax import lax
from jax.experimental import pallas as pl
from jax.experimental.pallas import tpu as pltpu
from jax.experimental.pallas import tpu_sc as plsc

_H = 256
_WG = 192
_F = 64
_HW = _H * _WG
_N = 1048576
_NC = 2
_NS = 16
_NW = _NC * _NS
_PPW = _N // _NW
_CHUNK = 2048
_NCHUNK = _PPW // _CHUNK
_VPC = _CHUNK // 16


_HB = 32


def _proj_body(g_ref, w_ref, b_ref, p_ref):
    g = g_ref[...].reshape(_HB * _WG, _F)
    p_ref[...] = (
        jnp.dot(g, w_ref[...], preferred_element_type=jnp.float32)
        + b_ref[...]
    )


def _project(grid3d, w_pad, b_pad):
    return pl.pallas_call(
        _proj_body,
        grid=(_H // _HB,),
        in_specs=[
            pl.BlockSpec((_HB, _WG, _F), lambda i: (i, 0, 0)),
            pl.BlockSpec((_F, 8), lambda i: (0, 0)),
            pl.BlockSpec((1, 8), lambda i: (0, 0)),
        ],
        out_specs=pl.BlockSpec((_HB * _WG, 8), lambda i: (i, 0)),
        out_shape=jax.ShapeDtypeStruct((_HW, 8), jnp.float32),
    )(grid3d, w_pad, b_pad)


def _interp_body(x_hbm, pa_hbm, pb_hbm, out_hbm, pa_v, pb_v, xs_v, os_v):
    wid = lax.axis_index("s") * _NC + lax.axis_index("c")
    pltpu.sync_copy(pa_hbm, pa_v)
    pltpu.sync_copy(pb_hbm, pb_v)
    base = wid * _PPW
    lane = lax.iota(jnp.int32, 16)
    hmask = jnp.int32(-65536)

    def vec_body(v, _):
        ix = (v * 16 + lane) * 2
        xv = plsc.load_gather(xs_v, [ix])
        yv = plsc.load_gather(xs_v, [ix + 1])
        gx = xv * jnp.float32(_H - 1)
        gy = yv * jnp.float32(_WG - 1)
        x0 = jnp.minimum(gx.astype(jnp.int32), _H - 2)
        y0 = jnp.minimum(gy.astype(jnp.int32), _WG - 2)
        wx = gx - x0.astype(jnp.float32)
        wy = gy - y0.astype(jnp.float32)
        i00 = x0 * _WG + y0
        i01 = i00 + 1
        i10 = i00 + _WG
        i11 = i00 + _WG + 1
        a00 = plsc.load_gather(pa_v, [i00])
        a01 = plsc.load_gather(pa_v, [i01])
        a10 = plsc.load_gather(pa_v, [i10])
        a11 = plsc.load_gather(pa_v, [i11])
        b00 = plsc.load_gather(pb_v, [i00])
        b01 = plsc.load_gather(pb_v, [i01])
        b10 = plsc.load_gather(pb_v, [i10])
        b11 = plsc.load_gather(pb_v, [i11])
        u = 1.0 - wx
        t = 1.0 - wy
        w00 = u * t
        w10 = wx * t
        w01 = u * wy
        w11 = wx * wy
        o0 = (w00 * plsc.bitcast(a00 << 16, jnp.float32)
              + w10 * plsc.bitcast(a10 << 16, jnp.float32)
              + w01 * plsc.bitcast(a01 << 16, jnp.float32)
              + w11 * plsc.bitcast(a11 << 16, jnp.float32))
        o1 = (w00 * plsc.bitcast(a00 & hmask, jnp.float32)
              + w10 * plsc.bitcast(a10 & hmask, jnp.float32)
              + w01 * plsc.bitcast(a01 & hmask, jnp.float32)
              + w11 * plsc.bitcast(a11 & hmask, jnp.float32))
        o2 = w00 * b00 + w10 * b10 + w01 * b01 + w11 * b11
        si = (v * 16 + lane) * 3
        plsc.store_scatter(os_v, [si], o0)
        plsc.store_scatter(os_v, [si + 1], o1)
        plsc.store_scatter(os_v, [si + 2], o2)
        return 0

    def chunk_body(j, _):
        cbase = base + j * _CHUNK
        pltpu.sync_copy(x_hbm.at[pl.ds(cbase * 2, _CHUNK * 2)], xs_v)
        lax.fori_loop(0, _VPC, vec_body, 0)
        pltpu.sync_copy(os_v, out_hbm.at[pl.ds(cbase * 3, _CHUNK * 3)])
        return 0

    lax.fori_loop(0, _NCHUNK, chunk_body, 0)


_interp = functools.partial(
    pl.kernel,
    out_type=jax.ShapeDtypeStruct((_N * 3,), jnp.float32),
    mesh=plsc.VectorSubcoreMesh(
        core_axis_name="c", subcore_axis_name="s",
        num_cores=_NC, num_subcores=_NS,
    ),
    scratch_types=[
        pltpu.VMEM((_HW,), jnp.int32),
        pltpu.VMEM((_HW,), jnp.float32),
        pltpu.VMEM((_CHUNK * 2,), jnp.float32),
        pltpu.VMEM((_CHUNK * 3,), jnp.float32),
    ],
    compiler_params=pltpu.CompilerParams(needs_layout_passes=False),
)(_interp_body)


def kernel(x, grid, W_mat, b):
    w_pad = jnp.zeros((_F, 8), jnp.float32).at[:, :3].set(W_mat)
    b_pad = jnp.zeros((1, 8), jnp.float32).at[0, :3].set(b)
    p = _project(grid, w_pad, b_pad)
    u0 = lax.bitcast_convert_type(
        p[:, 0].astype(jnp.bfloat16), jnp.uint16).astype(jnp.uint32)
    u1 = lax.bitcast_convert_type(
        p[:, 1].astype(jnp.bfloat16), jnp.uint16).astype(jnp.uint32)
    plane_a = lax.bitcast_convert_type((u1 << 16) | u0, jnp.int32)
    out_flat = _interp(x.reshape(_N * 2), plane_a, p[:, 2])
    return (out_flat.reshape(_N, 3), x)
